# baseline (speedup 1.0000x reference)
import functools

import jax
import jax.numpy as jnp
from jax import lax
from jax.experimental import pallas as pl
from jax.experimental.pallas import tpu as pltpu
from jax.experimental.pallas import tpu_sc as plsc

_CH = 128
_TW = 256
_MW = 128


def _softplus(x):
    return jnp.maximum(x, 0.0) + jnp.log1p(jnp.exp(-jnp.abs(x)))


def _lnorm(x, g, b, eps=1e-5):
    m = jnp.mean(x, axis=-1, keepdims=True)
    v = jnp.mean((x - m) ** 2, axis=-1, keepdims=True)
    return (x - m) / jnp.sqrt(v + eps) * g + b


def _prep(h, pvsrc, Wd, Ws, bn=2000):
    N, F = h.shape

    def kern(h_ref, pv_ref, wd_ref, ws_ref, p_ref, q_ref):
        hb = h_ref[...]
        pv4 = pv_ref[...][:, :4]
        z = jnp.zeros((hb.shape[0], _TW - 132), jnp.float32)
        dp = jnp.dot(hb, wd_ref[...], preferred_element_type=jnp.float32)
        dq = jnp.dot(hb, ws_ref[...], preferred_element_type=jnp.float32)
        p_ref[...] = jnp.concatenate([dp, -pv4, z], axis=1)
        q_ref[...] = jnp.concatenate([dq, pv4, z], axis=1)

    Fp = pvsrc.shape[1]
    return pl.pallas_call(
        kern,
        grid=(N // bn,),
        in_specs=[
            pl.BlockSpec((bn, F), lambda i: (i, 0)),
            pl.BlockSpec((bn, Fp), lambda i: (i, 0)),
            pl.BlockSpec((F, 128), lambda i: (0, 0)),
            pl.BlockSpec((F, 128), lambda i: (0, 0)),
        ],
        out_specs=[pl.BlockSpec((bn, _TW), lambda i: (i, 0))] * 2,
        out_shape=[jax.ShapeDtypeStruct((N, _TW), jnp.float32)] * 2,
    )(h, pvsrc, Wd, Ws)


_GW = 144
_GCH = 80


def _sc_gather(P, Q, src, dst):
    E = src.shape[0]
    info = plsc.get_sparse_core_info()
    NC, NS = info.num_cores, info.num_subcores
    NW = NC * NS
    n_chunks = E // _GCH
    kpt = (n_chunks + NW - 1) // NW
    mesh = plsc.VectorSubcoreMesh(core_axis_name="c", subcore_axis_name="s")

    @functools.partial(
        pl.kernel,
        mesh=mesh,
        out_type=(jax.ShapeDtypeStruct((E, 128), jnp.float32),
                  jax.ShapeDtypeStruct((E, 16), jnp.float32)),
        scratch_types=[
            pltpu.VMEM((_GCH,), jnp.int32),
            pltpu.VMEM((_GCH,), jnp.int32),
            pltpu.VMEM((_GCH,), jnp.int32),
            pltpu.VMEM((_GCH,), jnp.int32),
            pltpu.VMEM((_GCH, _TW), jnp.float32),
            pltpu.VMEM((_GCH, _TW), jnp.float32),
            pltpu.VMEM((_GCH, _TW), jnp.float32),
            pltpu.VMEM((_GCH, _TW), jnp.float32),
            pltpu.VMEM((_GCH, 16), jnp.float32),
            pltpu.VMEM((_GCH, 16), jnp.float32),
            pltpu.SemaphoreType.DMA,
            pltpu.SemaphoreType.DMA,
        ],
    )
    def k(p_hbm, q_hbm, src_hbm, dst_hbm, pre_hbm, rel_hbm,
          di0, si0, di1, si1, pg0, qg0, pg1, qg1, rl0, rl1, sem0, sem1):
        wid = lax.axis_index("s") * NC + lax.axis_index("c")
        di = (di0, di1)
        si = (si0, si1)
        pg = (pg0, pg1)
        qg = (qg0, qg1)
        rl = (rl0, rl1)
        sem = (sem0, sem1)

        def fetch(k_idx, b):
            base = (wid + k_idx * NW) * _GCH
            pltpu.sync_copy(dst_hbm.at[pl.ds(base, _GCH)], di[b])
            pltpu.sync_copy(src_hbm.at[pl.ds(base, _GCH)], si[b])
            pltpu.async_copy(p_hbm.at[di[b]], pg[b], sem[b])
            pltpu.async_copy(q_hbm.at[si[b]], qg[b], sem[b])

        def process(k_idx, b):
            pltpu.make_async_copy(p_hbm.at[pl.ds(0, _GCH)], pg[b],
                                  sem[b]).wait()
            pltpu.make_async_copy(q_hbm.at[pl.ds(0, _GCH)], qg[b],
                                  sem[b]).wait()

            def addr(r, a):
                for cc in range(8):
                    s = pl.ds(cc * 16, 16)
                    pg[b][r, s] = pg[b][r, s] + qg[b][r, s]
                s = pl.ds(128, 16)
                rl[b][r, pl.ds(0, 16)] = pg[b][r, s] + qg[b][r, s]
                return a

            lax.fori_loop(0, _GCH, addr, 0)
            base = (wid + k_idx * NW) * _GCH
            pltpu.sync_copy(pg[b].at[:, pl.ds(0, 128)],
                            pre_hbm.at[pl.ds(base, _GCH)])
            pltpu.sync_copy(rl[b], rel_hbm.at[pl.ds(base, _GCH)])

        def have(k_idx):
            return wid + k_idx * NW < n_chunks

        @pl.when(have(0))
        def _():
            fetch(0, 0)

        def pair(kk, carry):
            k0 = kk * 2
            k1 = k0 + 1

            @pl.when(have(k0))
            def _():
                @pl.when(have(k1))
                def _():
                    fetch(k1, 1)
                process(k0, 0)

            @pl.when(have(k1))
            def _():
                @pl.when(have(k1 + 1))
                def _():
                    fetch(k1 + 1, 0)
                process(k1, 1)

            return carry

        lax.fori_loop(0, (kpt + 1) // 2, pair, 0)

    return k(P, Q, src, dst)


_NP = 10240


def _sc_scatter(msgs, dsts, N):
    E = dsts[0].shape[0]
    nsp = len(msgs)
    info = plsc.get_sparse_core_info()
    NC, NS = info.num_cores, info.num_subcores
    NW = NC * NS
    n_chunks = E // _CH
    maxj = (n_chunks + NW - 1) // NW
    rpt = _NP // NS
    wbr = 160
    mesh = plsc.VectorSubcoreMesh(core_axis_name="c", subcore_axis_name="s")

    @functools.partial(
        pl.kernel,
        mesh=mesh,
        out_type=jax.ShapeDtypeStruct((NC * _NP, _MW), jnp.float32),
        scratch_types=[
            pltpu.VMEM((_CH,), jnp.int32),
            pltpu.VMEM((_CH, _MW), jnp.float32),
            pltpu.VMEM((wbr, _MW), jnp.float32),
            pltpu.VMEM_SHARED((_NP, _MW), jnp.float32),
        ],
    )
    def k(*refs):
        msg_hbms = refs[:nsp]
        dst_hbms = refs[nsp:2 * nsp]
        out_hbm = refs[2 * nsp]
        idx_v, msg_v, wb_v, acc_sh = refs[2 * nsp + 1:]
        cid = lax.axis_index("c")
        sid = lax.axis_index("s")
        wid = sid * NC + cid

        def zr(r, a):
            for cc in range(_MW // 16):
                wb_v[r, pl.ds(cc * 16, 16)] = jnp.zeros((16,), jnp.float32)
            return a

        lax.fori_loop(0, wbr, zr, 0)
        for ph in range(rpt // wbr):
            pltpu.sync_copy(wb_v, acc_sh.at[pl.ds(sid * rpt + ph * wbr, wbr)])
        plsc.subcore_barrier()

        for msg_hbm, dst_hbm in zip(msg_hbms, dst_hbms):
            def chunk(j, carry, m=msg_hbm, d=dst_hbm):
                c = wid + j * NW

                @pl.when(c < n_chunks)
                def _():
                    base = c * _CH
                    pltpu.sync_copy(d.at[pl.ds(base, _CH)], idx_v)
                    pltpu.sync_copy(m.at[pl.ds(base, _CH)], msg_v)
                    pltpu.sync_copy(msg_v, acc_sh.at[idx_v], add=True)

                return carry

            lax.fori_loop(0, maxj, chunk, 0)
        plsc.subcore_barrier()
        for ph in range(rpt // wbr):
            pltpu.sync_copy(acc_sh.at[pl.ds(sid * rpt + ph * wbr, wbr)], wb_v)
            pltpu.sync_copy(
                wb_v, out_hbm.at[pl.ds(cid * _NP + sid * rpt + ph * wbr, wbr)])

    return k(*msgs, *dsts)


def _edge(pre, rel, p, be=4000):
    E = pre.shape[0]
    fin = (p["phi_e"]["l1"]["W"].shape[0] - 5) // 2
    W1e = p["phi_e"]["l1"]["W"]
    W1v = p["phi_v"]["l1"]["W"]
    z3 = jnp.zeros((3, 64), jnp.float32)
    Wse = jnp.concatenate([W1e[2 * fin:2 * fin + 5], z3], axis=0)
    Wsv = jnp.concatenate([W1v[2 * fin:2 * fin + 5], z3], axis=0)
    r1 = lambda a: a.reshape(1, -1)
    args = (
        pre, rel, Wse, r1(p["phi_e"]["l1"]["b"]), r1(p["phi_e"]["g"]),
        r1(p["phi_e"]["be"]), p["phi_e"]["l2"]["W"], r1(p["phi_e"]["l2"]["b"]),
        p["phi_e"]["l3"]["W"], r1(p["phi_e"]["l3"]["b"]), Wsv,
        r1(p["phi_v"]["l1"]["b"]), r1(p["phi_v"]["g"]), r1(p["phi_v"]["be"]),
        r1(p["phi_v"]["l2"]["W"][:, 0]),
        jnp.full((1, 64), p["phi_v"]["l2"]["b"][0], jnp.float32),
    )

    def kern(pre_ref, rel_ref, wse, bse, ge, bee, we2, be2, we3, be3,
             wsv, bsv, gv, bev, wv2, bv2, out_ref):
        pre_b = pre_ref[...]
        rel_b = rel_ref[...]
        rel = rel_b[:, 0:2]
        rv = rel_b[:, 2:4]
        ds = jnp.sum(rel * rel, axis=-1, keepdims=True)
        dv = jnp.sum(rv * rel, axis=-1, keepdims=True)
        r2 = jnp.minimum(1.0 / (ds + 0.05), 20.0)
        r6 = jnp.minimum(r2 ** 3, 400.0)
        r12 = jnp.minimum(r6 ** 2, 160000.0)
        zc = jnp.zeros_like(ds)
        scal = jnp.concatenate([ds, dv, r2, r6, r12, zc, zc, zc], axis=1)
        sc_e = jnp.dot(scal, wse[...], preferred_element_type=jnp.float32)
        he = pre_b[:, :64] + sc_e + bse[...]
        he = _softplus(_lnorm(he, ge[...], bee[...]))
        he = _softplus(jnp.dot(he, we2[...], preferred_element_type=jnp.float32)
                       + be2[...])
        mh = jnp.dot(he, we3[...], preferred_element_type=jnp.float32) + be3[...]
        sc_v = jnp.dot(scal, wsv[...], preferred_element_type=jnp.float32)
        hv = pre_b[:, 64:128] + sc_v + bsv[...]
        hv = _softplus(_lnorm(hv, gv[...], bev[...]))
        sv = jnp.sum(hv * wv2[...], axis=-1, keepdims=True) + bv2[:, 0:1]
        mv = sv * rel
        one = jnp.ones_like(ds)
        pad = jnp.zeros((pre_b.shape[0], _MW - 67), jnp.float32)
        out_ref[...] = jnp.concatenate([mh, mv, one, pad], axis=1)

    wspec = lambda a: pl.BlockSpec(a.shape, lambda i: tuple(0 for _ in a.shape))
    return pl.pallas_call(
        kern,
        grid=(E // be,),
        in_specs=[pl.BlockSpec((be, 128), lambda i: (i, 0)),
                  pl.BlockSpec((be, 16), lambda i: (i, 0))]
        + [wspec(a) for a in args[2:]],
        out_specs=pl.BlockSpec((be, _MW), lambda i: (i, 0)),
        out_shape=jax.ShapeDtypeStruct((E, _MW), jnp.float32),
    )(*args)


def _node(parts, h_in, p, ln, prep_w=None, bn=2000):
    N, fin = h_in.shape
    Wh1 = p["phi_h"]["l1"]["W"]
    with_sc = "sc" in p
    r1 = lambda a: a.reshape(1, -1)
    args = list(parts) + [h_in, Wh1[:fin], Wh1[fin:fin + 64], Wh1[fin + 64:fin + 65],
        r1(p["phi_h"]["l1"]["b"]), r1(p["phi_h"]["g"]), r1(p["phi_h"]["be"]),
        p["phi_h"]["l2"]["W"], r1(p["phi_h"]["l2"]["b"]),
        r1(ln["g"]), r1(ln["b"]),
    ]
    if with_sc:
        args += [p["sc"]["W"], r1(p["sc"]["b"])]
    if prep_w is not None:
        args += list(prep_w)
    fout = p["phi_h"]["l2"]["W"].shape[1]
    npart = len(parts)

    def kern(*refs):
        it = iter(refs)
        nxt = lambda: next(it)
        prefs = [nxt() for _ in range(npart)]
        h_ref, wa, wb, wc, bh1, gh, bh, wh2, bh2, lng, lnb = (
            nxt() for _ in range(11))
        if with_sc:
            wsc, bsc = nxt(), nxt()
        if prep_w is not None:
            wd, ws = nxt(), nxt()
        outs = list(it)
        acc = prefs[0][...]
        for pr in prefs[1:]:
            acc = acc + pr[...]
        mh = acc[:, :64]
        mv = acc[:, 64:66]
        cnt = acc[:, 66:67]
        den = jnp.maximum(cnt, 1.0)
        mh = mh / den
        mv = mv / den
        mvn = jnp.sqrt(jnp.sum((mv + 1e-8) ** 2, axis=-1, keepdims=True))
        hb = h_ref[...]
        t = (jnp.dot(hb, wa[...], preferred_element_type=jnp.float32)
             + jnp.dot(mh, wb[...], preferred_element_type=jnp.float32)
             + mvn * wc[...] + bh1[...])
        t = _softplus(_lnorm(t, gh[...], bh[...]))
        up = jnp.dot(t, wh2[...], preferred_element_type=jnp.float32) + bh2[...]
        if with_sc:
            short = jnp.dot(hb, wsc[...], preferred_element_type=jnp.float32) \
                + bsc[...]
        else:
            short = hb
        h_new = _lnorm(jnp.maximum(short + up, 0.0), lng[...], lnb[...])
        outs[0][...] = h_new
        if prep_w is not None:
            pv4 = hb[:, :4]
            z = jnp.zeros((h_new.shape[0], _TW - 132), jnp.float32)
            dp = jnp.dot(h_new, wd[...], preferred_element_type=jnp.float32)
            dq = jnp.dot(h_new, ws[...], preferred_element_type=jnp.float32)
            outs[1][...] = jnp.concatenate([dp, -pv4, z], axis=1)
            outs[2][...] = jnp.concatenate([dq, pv4, z], axis=1)

    wspec = lambda a: pl.BlockSpec(a.shape, lambda i: tuple(0 for _ in a.shape))
    in_specs = [pl.BlockSpec((bn, _MW), lambda i: (i, 0))] * npart + [
        pl.BlockSpec((bn, fin), lambda i: (i, 0)),
    ] + [wspec(a) for a in args[npart + 1:]]
    out_shape = [jax.ShapeDtypeStruct((N, fout), jnp.float32)]
    out_specs = [pl.BlockSpec((bn, fout), lambda i: (i, 0))]
    if prep_w is not None:
        out_shape += [jax.ShapeDtypeStruct((N, _TW), jnp.float32)] * 2
        out_specs += [pl.BlockSpec((bn, _TW), lambda i: (i, 0))] * 2
    return pl.pallas_call(
        kern,
        grid=(N // bn,),
        in_specs=in_specs,
        out_specs=out_specs,
        out_shape=out_shape,
    )(*args)


def _pool(h, pv, batch2d, params, bn=2000):
    N = h.shape[0]
    ngrid = N // bn
    r1 = lambda a: a.reshape(1, -1)
    args = (
        h, pv, batch2d, params["pool"]["W"], r1(params["pool"]["b"]),
        params["out1"]["W"], r1(params["out1"]["b"]),
        params["out2"]["W"], r1(params["out2"]["b"]),
        r1(params["latent_gain"]),
    )

    def kern(h_ref, pv_ref, b_ref, wp, bp, wo1, bo1, wo2, bo2, gain,
             s_ref, den_ref, pooled_ref, mu_ref, loss_ref, lat_ref):
        i = pl.program_id(0)
        hb = h_ref[...]
        logits = jnp.dot(hb, wp[...], preferred_element_type=jnp.float32) \
            + bp[...]
        logits = logits - jnp.max(logits, axis=-1, keepdims=True)
        es = jnp.exp(logits)
        s = es / jnp.sum(es, axis=-1, keepdims=True)
        s_ref[...] = s
        bb = b_ref[...]
        gids = lax.broadcasted_iota(jnp.int32, (1, 8), 1)
        oh = (bb == gids).astype(jnp.float32)
        ones8 = jnp.ones((s.shape[0], 8), jnp.float32)
        pvb = pv_ref[...]

        @pl.when(i == 0)
        def _():
            den_ref[...] = jnp.zeros_like(den_ref)
            pooled_ref[...] = jnp.zeros_like(pooled_ref)
            mu_ref[...] = jnp.zeros_like(mu_ref)
            loss_ref[...] = jnp.zeros_like(loss_ref)

        loss_ref[...] += jnp.sum(s * jnp.log(s + 1e-8), axis=0, keepdims=True)
        for g in range(8):
            ms = s * oh[:, g:g + 1]
            pc = lax.dot_general(ms, hb, (((0,), (0,)), ((), ())),
                                 preferred_element_type=jnp.float32)
            mc = lax.dot_general(ms, pvb, (((0,), (0,)), ((), ())),
                                 preferred_element_type=jnp.float32)
            dc = lax.dot_general(ms, ones8, (((0,), (0,)), ((), ())),
                                 preferred_element_type=jnp.float32)
            pooled_ref[pl.ds(g * 16, 16), :] += pc
            mu_ref[pl.ds(g * 16, 16), :] += mc
            den_ref[pl.ds(g * 16, 16), :] += dc

        @pl.when(i == ngrid - 1)
        def _():
            den = den_ref[...][:, 0:1]
            pm = pooled_ref[...] / (den + 1e-8)
            mu_ref[...] = mu_ref[...] / (den + 1e-8)
            z = jnp.maximum(
                jnp.dot(pm, wo1[...], preferred_element_type=jnp.float32)
                + bo1[...], 0.0)
            latv = jnp.dot(z, wo2[...], preferred_element_type=jnp.float32) \
                + bo2[...]
            latv = latv * gain[...]
            m = jnp.mean(latv, axis=-1, keepdims=True)
            v = jnp.mean((latv - m) ** 2, axis=-1, keepdims=True)
            lat_ref[...] = (latv - m) / jnp.sqrt(v + 1e-5)

    wspec = lambda a: pl.BlockSpec(a.shape, lambda i: tuple(0 for _ in a.shape))
    czero = lambda shape: pl.BlockSpec(shape, lambda i: tuple(0 for _ in shape))
    return pl.pallas_call(
        kern,
        grid=(ngrid,),
        in_specs=[
            pl.BlockSpec((bn, 64), lambda i: (i, 0)),
            pl.BlockSpec((bn, 16), lambda i: (i, 0)),
            pl.BlockSpec((bn, 1), lambda i: (i, 0)),
        ] + [wspec(a) for a in args[3:]],
        out_specs=[
            pl.BlockSpec((bn, 16), lambda i: (i, 0)),
            czero((128, 8)),
            czero((128, 64)),
            czero((128, 16)),
            czero((1, 16)),
            czero((128, 32)),
        ],
        out_shape=[
            jax.ShapeDtypeStruct((N, 16), jnp.float32),
            jax.ShapeDtypeStruct((128, 8), jnp.float32),
            jax.ShapeDtypeStruct((128, 64), jnp.float32),
            jax.ShapeDtypeStruct((128, 16), jnp.float32),
            jax.ShapeDtypeStruct((1, 16), jnp.float32),
            jax.ShapeDtypeStruct((128, 32), jnp.float32),
        ],
    )(*args)


def _layer_tables(p, fin):
    W1e = p["phi_e"]["l1"]["W"]
    W1v = p["phi_v"]["l1"]["W"]
    Wd = jnp.concatenate([W1e[:fin], W1v[:fin]], axis=1)
    Ws = jnp.concatenate([W1e[fin:2 * fin], W1v[fin:2 * fin]], axis=1)
    return Wd, Ws


def kernel(x, params, edge_index, batch):
    src = edge_index[0]
    dst = edge_index[1]
    N, F = x.shape
    p1 = params["gnn1"]
    p2 = params["gnn2"]

    nsp = 4
    Eh = src.shape[0] // nsp
    halves = tuple((src[i * Eh:(i + 1) * Eh], dst[i * Eh:(i + 1) * Eh])
                   for i in range(nsp))

    def layer(P, Q, pp):
        msgs = []
        for s_h, d_h in halves:
            pre_h, rel_h = _sc_gather(P, Q, s_h, d_h)
            msgs.append(_edge(pre_h, rel_h, pp))
        ph = _sc_scatter(msgs, [d for _, d in halves], N)
        return [ph[:N], ph[_NP:_NP + N]]

    Wd1, Ws1 = _layer_tables(p1, F)
    P1, Q1 = _prep(x, x, Wd1, Ws1)
    Wd2, Ws2 = _layer_tables(p2, 64)
    h1, P2, Q2 = _node(layer(P1, Q1, p1), x, p1, params["ln1"],
                       prep_w=(Wd2, Ws2))
    (h2,) = _node(layer(P2, Q2, p2), h1, p2, params["ln2"])

    pvp = jnp.pad(x[:, :4], ((0, 0), (0, 12)))
    s_out, _den, _pooled, mu2d, loss, lat2d = _pool(
        h2, pvp, batch.reshape(N, 1).astype(jnp.int32), params)
    latent = lat2d.reshape(8, 16, 32)
    mu = mu2d[:, :2].reshape(8, 16, 2)
    assign_losses = -jnp.sum(loss) / N
    return latent, s_out, assign_losses, mu


# speedup vs baseline: 1.1048x; 1.1048x over previous
import functools

import jax
import jax.numpy as jnp
from jax import lax
from jax.experimental import pallas as pl
from jax.experimental.pallas import tpu as pltpu
from jax.experimental.pallas import tpu_sc as plsc

_CH = 128
_TW = 256
_MW = 128


def _softplus(x):
    return jnp.maximum(x, 0.0) + jnp.log1p(jnp.exp(-jnp.abs(x)))


def _lnorm(x, g, b, eps=1e-5):
    m = jnp.mean(x, axis=-1, keepdims=True)
    v = jnp.mean((x - m) ** 2, axis=-1, keepdims=True)
    return (x - m) / jnp.sqrt(v + eps) * g + b


def _prep(h, pvsrc, Wd, Ws, bn=2000):
    N, F = h.shape

    def kern(h_ref, pv_ref, wd_ref, ws_ref, p_ref, q_ref):
        hb = h_ref[...]
        pv4 = pv_ref[...][:, :4]
        z = jnp.zeros((hb.shape[0], _TW - 132), jnp.float32)
        dp = jnp.dot(hb, wd_ref[...], preferred_element_type=jnp.float32)
        dq = jnp.dot(hb, ws_ref[...], preferred_element_type=jnp.float32)
        p_ref[...] = jnp.concatenate([dp, -pv4, z], axis=1)
        q_ref[...] = jnp.concatenate([dq, pv4, z], axis=1)

    Fp = pvsrc.shape[1]
    return pl.pallas_call(
        kern,
        grid=(N // bn,),
        in_specs=[
            pl.BlockSpec((bn, F), lambda i: (i, 0)),
            pl.BlockSpec((bn, Fp), lambda i: (i, 0)),
            pl.BlockSpec((F, 128), lambda i: (0, 0)),
            pl.BlockSpec((F, 128), lambda i: (0, 0)),
        ],
        out_specs=[pl.BlockSpec((bn, _TW), lambda i: (i, 0))] * 2,
        out_shape=[jax.ShapeDtypeStruct((N, _TW), jnp.float32)] * 2,
    )(h, pvsrc, Wd, Ws)


_GW = 144
_GCH = 80


def _sc_gather(P, Q, src, dst):
    E = src.shape[0]
    info = plsc.get_sparse_core_info()
    NC, NS = info.num_cores, info.num_subcores
    NW = NC * NS
    n_chunks = E // _GCH
    kpt = (n_chunks + NW - 1) // NW
    mesh = plsc.VectorSubcoreMesh(core_axis_name="c", subcore_axis_name="s")

    @functools.partial(
        pl.kernel,
        mesh=mesh,
        out_type=(jax.ShapeDtypeStruct((E, 128), jnp.float32),
                  jax.ShapeDtypeStruct((E, 16), jnp.float32)),
        scratch_types=[
            pltpu.VMEM((_GCH,), jnp.int32),
            pltpu.VMEM((_GCH,), jnp.int32),
            pltpu.VMEM((_GCH,), jnp.int32),
            pltpu.VMEM((_GCH,), jnp.int32),
            pltpu.VMEM((_GCH, _TW), jnp.float32),
            pltpu.VMEM((_GCH, _TW), jnp.float32),
            pltpu.VMEM((_GCH, _TW), jnp.float32),
            pltpu.VMEM((_GCH, _TW), jnp.float32),
            pltpu.VMEM((_GCH, 16), jnp.float32),
            pltpu.VMEM((_GCH, 16), jnp.float32),
            pltpu.SemaphoreType.DMA,
            pltpu.SemaphoreType.DMA,
        ],
    )
    def k(p_hbm, q_hbm, src_hbm, dst_hbm, pre_hbm, rel_hbm,
          di0, si0, di1, si1, pg0, qg0, pg1, qg1, rl0, rl1, sem0, sem1):
        wid = lax.axis_index("s") * NC + lax.axis_index("c")
        di = (di0, di1)
        si = (si0, si1)
        pg = (pg0, pg1)
        qg = (qg0, qg1)
        rl = (rl0, rl1)
        sem = (sem0, sem1)

        def fetch(k_idx, b):
            base = (wid + k_idx * NW) * _GCH
            pltpu.sync_copy(dst_hbm.at[pl.ds(base, _GCH)], di[b])
            pltpu.sync_copy(src_hbm.at[pl.ds(base, _GCH)], si[b])
            pltpu.async_copy(p_hbm.at[di[b]], pg[b], sem[b])
            pltpu.async_copy(q_hbm.at[si[b]], qg[b], sem[b])

        def process(k_idx, b):
            pltpu.make_async_copy(p_hbm.at[pl.ds(0, _GCH)], pg[b],
                                  sem[b]).wait()
            pltpu.make_async_copy(q_hbm.at[pl.ds(0, _GCH)], qg[b],
                                  sem[b]).wait()

            def addr(r, a):
                for cc in range(8):
                    s = pl.ds(cc * 16, 16)
                    pg[b][r, s] = pg[b][r, s] + qg[b][r, s]
                s = pl.ds(128, 16)
                rl[b][r, pl.ds(0, 16)] = pg[b][r, s] + qg[b][r, s]
                return a

            lax.fori_loop(0, _GCH, addr, 0)
            base = (wid + k_idx * NW) * _GCH
            pltpu.sync_copy(pg[b].at[:, pl.ds(0, 128)],
                            pre_hbm.at[pl.ds(base, _GCH)])
            pltpu.sync_copy(rl[b], rel_hbm.at[pl.ds(base, _GCH)])

        def have(k_idx):
            return wid + k_idx * NW < n_chunks

        @pl.when(have(0))
        def _():
            fetch(0, 0)

        def pair(kk, carry):
            k0 = kk * 2
            k1 = k0 + 1

            @pl.when(have(k0))
            def _():
                @pl.when(have(k1))
                def _():
                    fetch(k1, 1)
                process(k0, 0)

            @pl.when(have(k1))
            def _():
                @pl.when(have(k1 + 1))
                def _():
                    fetch(k1 + 1, 0)
                process(k1, 1)

            return carry

        lax.fori_loop(0, (kpt + 1) // 2, pair, 0)

    return k(P, Q, src, dst)


_NP = 10240


def _sc_scatter(msgs, dsts, N):
    E = dsts[0].shape[0]
    nsp = len(msgs)
    info = plsc.get_sparse_core_info()
    NC, NS = info.num_cores, info.num_subcores
    NW = NC * NS
    n_chunks = E // _CH
    maxj = (n_chunks + NW - 1) // NW
    rpt = _NP // NS
    wbr = 160
    mesh = plsc.VectorSubcoreMesh(core_axis_name="c", subcore_axis_name="s")

    @functools.partial(
        pl.kernel,
        mesh=mesh,
        out_type=jax.ShapeDtypeStruct((NC * _NP, _MW), jnp.float32),
        scratch_types=[
            pltpu.VMEM((_CH,), jnp.int32),
            pltpu.VMEM((_CH, _MW), jnp.float32),
            pltpu.VMEM((wbr, _MW), jnp.float32),
            pltpu.VMEM_SHARED((_NP, _MW), jnp.float32),
        ],
    )
    def k(*refs):
        msg_hbms = refs[:nsp]
        dst_hbms = refs[nsp:2 * nsp]
        out_hbm = refs[2 * nsp]
        idx_v, msg_v, wb_v, acc_sh = refs[2 * nsp + 1:]
        cid = lax.axis_index("c")
        sid = lax.axis_index("s")
        wid = sid * NC + cid

        def zr(r, a):
            for cc in range(_MW // 16):
                wb_v[r, pl.ds(cc * 16, 16)] = jnp.zeros((16,), jnp.float32)
            return a

        lax.fori_loop(0, wbr, zr, 0)
        for ph in range(rpt // wbr):
            pltpu.sync_copy(wb_v, acc_sh.at[pl.ds(sid * rpt + ph * wbr, wbr)])
        plsc.subcore_barrier()

        for msg_hbm, dst_hbm in zip(msg_hbms, dst_hbms):
            def chunk(j, carry, m=msg_hbm, d=dst_hbm):
                c = wid + j * NW

                @pl.when(c < n_chunks)
                def _():
                    base = c * _CH
                    pltpu.sync_copy(d.at[pl.ds(base, _CH)], idx_v)
                    pltpu.sync_copy(m.at[pl.ds(base, _CH)], msg_v)
                    pltpu.sync_copy(msg_v, acc_sh.at[idx_v], add=True)

                return carry

            lax.fori_loop(0, maxj, chunk, 0)
        plsc.subcore_barrier()
        for ph in range(rpt // wbr):
            pltpu.sync_copy(acc_sh.at[pl.ds(sid * rpt + ph * wbr, wbr)], wb_v)
            pltpu.sync_copy(
                wb_v, out_hbm.at[pl.ds(cid * _NP + sid * rpt + ph * wbr, wbr)])

    return k(*msgs, *dsts)


def _edge(pre, rel, p, be=4000):
    E = pre.shape[0]
    fin = (p["phi_e"]["l1"]["W"].shape[0] - 5) // 2
    W1e = p["phi_e"]["l1"]["W"]
    W1v = p["phi_v"]["l1"]["W"]
    z3 = jnp.zeros((3, 64), jnp.float32)
    Wse = jnp.concatenate([W1e[2 * fin:2 * fin + 5], z3], axis=0)
    Wsv = jnp.concatenate([W1v[2 * fin:2 * fin + 5], z3], axis=0)
    r1 = lambda a: a.reshape(1, -1)
    args = (
        pre, rel, Wse, r1(p["phi_e"]["l1"]["b"]), r1(p["phi_e"]["g"]),
        r1(p["phi_e"]["be"]), p["phi_e"]["l2"]["W"], r1(p["phi_e"]["l2"]["b"]),
        p["phi_e"]["l3"]["W"], r1(p["phi_e"]["l3"]["b"]), Wsv,
        r1(p["phi_v"]["l1"]["b"]), r1(p["phi_v"]["g"]), r1(p["phi_v"]["be"]),
        r1(p["phi_v"]["l2"]["W"][:, 0]),
        jnp.full((1, 64), p["phi_v"]["l2"]["b"][0], jnp.float32),
    )

    def kern(pre_ref, rel_ref, wse, bse, ge, bee, we2, be2, we3, be3,
             wsv, bsv, gv, bev, wv2, bv2, out_ref):
        pre_b = pre_ref[...]
        rel_b = rel_ref[...]
        rel = rel_b[:, 0:2]
        rv = rel_b[:, 2:4]
        ds = jnp.sum(rel * rel, axis=-1, keepdims=True)
        dv = jnp.sum(rv * rel, axis=-1, keepdims=True)
        r2 = jnp.minimum(1.0 / (ds + 0.05), 20.0)
        r6 = jnp.minimum(r2 ** 3, 400.0)
        r12 = jnp.minimum(r6 ** 2, 160000.0)
        zc = jnp.zeros_like(ds)
        scal = jnp.concatenate([ds, dv, r2, r6, r12, zc, zc, zc], axis=1)
        sc_e = jnp.dot(scal, wse[...], preferred_element_type=jnp.float32)
        he = pre_b[:, :64] + sc_e + bse[...]
        he = _softplus(_lnorm(he, ge[...], bee[...]))
        he = _softplus(jnp.dot(he, we2[...], preferred_element_type=jnp.float32)
                       + be2[...])
        mh = jnp.dot(he, we3[...], preferred_element_type=jnp.float32) + be3[...]
        sc_v = jnp.dot(scal, wsv[...], preferred_element_type=jnp.float32)
        hv = pre_b[:, 64:128] + sc_v + bsv[...]
        hv = _softplus(_lnorm(hv, gv[...], bev[...]))
        sv = jnp.sum(hv * wv2[...], axis=-1, keepdims=True) + bv2[:, 0:1]
        mv = sv * rel
        one = jnp.ones_like(ds)
        pad = jnp.zeros((pre_b.shape[0], _MW - 67), jnp.float32)
        out_ref[...] = jnp.concatenate([mh, mv, one, pad], axis=1)

    wspec = lambda a: pl.BlockSpec(a.shape, lambda i: tuple(0 for _ in a.shape))
    return pl.pallas_call(
        kern,
        grid=(E // be,),
        in_specs=[pl.BlockSpec((be, 128), lambda i: (i, 0)),
                  pl.BlockSpec((be, 16), lambda i: (i, 0))]
        + [wspec(a) for a in args[2:]],
        out_specs=pl.BlockSpec((be, _MW), lambda i: (i, 0)),
        out_shape=jax.ShapeDtypeStruct((E, _MW), jnp.float32),
    )(*args)


def _node(parts, h_in, p, ln, prep_w=None, bn=2000):
    N, fin = h_in.shape
    Wh1 = p["phi_h"]["l1"]["W"]
    with_sc = "sc" in p
    r1 = lambda a: a.reshape(1, -1)
    args = list(parts) + [h_in, Wh1[:fin], Wh1[fin:fin + 64], Wh1[fin + 64:fin + 65],
        r1(p["phi_h"]["l1"]["b"]), r1(p["phi_h"]["g"]), r1(p["phi_h"]["be"]),
        p["phi_h"]["l2"]["W"], r1(p["phi_h"]["l2"]["b"]),
        r1(ln["g"]), r1(ln["b"]),
    ]
    if with_sc:
        args += [p["sc"]["W"], r1(p["sc"]["b"])]
    if prep_w is not None:
        args += list(prep_w)
    fout = p["phi_h"]["l2"]["W"].shape[1]
    npart = len(parts)

    def kern(*refs):
        it = iter(refs)
        nxt = lambda: next(it)
        prefs = [nxt() for _ in range(npart)]
        h_ref, wa, wb, wc, bh1, gh, bh, wh2, bh2, lng, lnb = (
            nxt() for _ in range(11))
        if with_sc:
            wsc, bsc = nxt(), nxt()
        if prep_w is not None:
            wd, ws = nxt(), nxt()
        outs = list(it)
        acc = prefs[0][...]
        for pr in prefs[1:]:
            acc = acc + pr[...]
        mh = acc[:, :64]
        mv = acc[:, 64:66]
        cnt = acc[:, 66:67]
        den = jnp.maximum(cnt, 1.0)
        mh = mh / den
        mv = mv / den
        mvn = jnp.sqrt(jnp.sum((mv + 1e-8) ** 2, axis=-1, keepdims=True))
        hb = h_ref[...]
        t = (jnp.dot(hb, wa[...], preferred_element_type=jnp.float32)
             + jnp.dot(mh, wb[...], preferred_element_type=jnp.float32)
             + mvn * wc[...] + bh1[...])
        t = _softplus(_lnorm(t, gh[...], bh[...]))
        up = jnp.dot(t, wh2[...], preferred_element_type=jnp.float32) + bh2[...]
        if with_sc:
            short = jnp.dot(hb, wsc[...], preferred_element_type=jnp.float32) \
                + bsc[...]
        else:
            short = hb
        h_new = _lnorm(jnp.maximum(short + up, 0.0), lng[...], lnb[...])
        outs[0][...] = h_new
        if prep_w is not None:
            pv4 = hb[:, :4]
            z = jnp.zeros((h_new.shape[0], _TW - 132), jnp.float32)
            dp = jnp.dot(h_new, wd[...], preferred_element_type=jnp.float32)
            dq = jnp.dot(h_new, ws[...], preferred_element_type=jnp.float32)
            outs[1][...] = jnp.concatenate([dp, -pv4, z], axis=1)
            outs[2][...] = jnp.concatenate([dq, pv4, z], axis=1)

    wspec = lambda a: pl.BlockSpec(a.shape, lambda i: tuple(0 for _ in a.shape))
    in_specs = [pl.BlockSpec((bn, _MW), lambda i: (i, 0))] * npart + [
        pl.BlockSpec((bn, fin), lambda i: (i, 0)),
    ] + [wspec(a) for a in args[npart + 1:]]
    out_shape = [jax.ShapeDtypeStruct((N, fout), jnp.float32)]
    out_specs = [pl.BlockSpec((bn, fout), lambda i: (i, 0))]
    if prep_w is not None:
        out_shape += [jax.ShapeDtypeStruct((N, _TW), jnp.float32)] * 2
        out_specs += [pl.BlockSpec((bn, _TW), lambda i: (i, 0))] * 2
    return pl.pallas_call(
        kern,
        grid=(N // bn,),
        in_specs=in_specs,
        out_specs=out_specs,
        out_shape=out_shape,
    )(*args)


def _pool(h, pv, batch2d, params, bn=2000):
    N = h.shape[0]
    ngrid = N // bn
    r1 = lambda a: a.reshape(1, -1)
    args = (
        h, pv, batch2d, params["pool"]["W"], r1(params["pool"]["b"]),
        params["out1"]["W"], r1(params["out1"]["b"]),
        params["out2"]["W"], r1(params["out2"]["b"]),
        r1(params["latent_gain"]),
    )

    def kern(h_ref, pv_ref, b_ref, wp, bp, wo1, bo1, wo2, bo2, gain,
             s_ref, den_ref, pooled_ref, mu_ref, loss_ref, lat_ref):
        i = pl.program_id(0)
        hb = h_ref[...]
        logits = jnp.dot(hb, wp[...], preferred_element_type=jnp.float32) \
            + bp[...]
        logits = logits - jnp.max(logits, axis=-1, keepdims=True)
        es = jnp.exp(logits)
        s = es / jnp.sum(es, axis=-1, keepdims=True)
        s_ref[...] = s
        bb = b_ref[...]
        gids = lax.broadcasted_iota(jnp.int32, (1, 8), 1)
        oh = (bb == gids).astype(jnp.float32)
        ones8 = jnp.ones((s.shape[0], 8), jnp.float32)
        pvb = pv_ref[...]

        @pl.when(i == 0)
        def _():
            den_ref[...] = jnp.zeros_like(den_ref)
            pooled_ref[...] = jnp.zeros_like(pooled_ref)
            mu_ref[...] = jnp.zeros_like(mu_ref)
            loss_ref[...] = jnp.zeros_like(loss_ref)

        loss_ref[...] += jnp.sum(s * jnp.log(s + 1e-8), axis=0, keepdims=True)
        for g in range(8):
            ms = s * oh[:, g:g + 1]
            pc = lax.dot_general(ms, hb, (((0,), (0,)), ((), ())),
                                 preferred_element_type=jnp.float32)
            mc = lax.dot_general(ms, pvb, (((0,), (0,)), ((), ())),
                                 preferred_element_type=jnp.float32)
            dc = lax.dot_general(ms, ones8, (((0,), (0,)), ((), ())),
                                 preferred_element_type=jnp.float32)
            pooled_ref[pl.ds(g * 16, 16), :] += pc
            mu_ref[pl.ds(g * 16, 16), :] += mc
            den_ref[pl.ds(g * 16, 16), :] += dc

        @pl.when(i == ngrid - 1)
        def _():
            den = den_ref[...][:, 0:1]
            pm = pooled_ref[...] / (den + 1e-8)
            mu_ref[...] = mu_ref[...] / (den + 1e-8)
            z = jnp.maximum(
                jnp.dot(pm, wo1[...], preferred_element_type=jnp.float32)
                + bo1[...], 0.0)
            latv = jnp.dot(z, wo2[...], preferred_element_type=jnp.float32) \
                + bo2[...]
            latv = latv * gain[...]
            m = jnp.mean(latv, axis=-1, keepdims=True)
            v = jnp.mean((latv - m) ** 2, axis=-1, keepdims=True)
            lat_ref[...] = (latv - m) / jnp.sqrt(v + 1e-5)

    wspec = lambda a: pl.BlockSpec(a.shape, lambda i: tuple(0 for _ in a.shape))
    czero = lambda shape: pl.BlockSpec(shape, lambda i: tuple(0 for _ in shape))
    return pl.pallas_call(
        kern,
        grid=(ngrid,),
        in_specs=[
            pl.BlockSpec((bn, 64), lambda i: (i, 0)),
            pl.BlockSpec((bn, 16), lambda i: (i, 0)),
            pl.BlockSpec((bn, 1), lambda i: (i, 0)),
        ] + [wspec(a) for a in args[3:]],
        out_specs=[
            pl.BlockSpec((bn, 16), lambda i: (i, 0)),
            czero((128, 8)),
            czero((128, 64)),
            czero((128, 16)),
            czero((1, 16)),
            czero((128, 32)),
        ],
        out_shape=[
            jax.ShapeDtypeStruct((N, 16), jnp.float32),
            jax.ShapeDtypeStruct((128, 8), jnp.float32),
            jax.ShapeDtypeStruct((128, 64), jnp.float32),
            jax.ShapeDtypeStruct((128, 16), jnp.float32),
            jax.ShapeDtypeStruct((1, 16), jnp.float32),
            jax.ShapeDtypeStruct((128, 32), jnp.float32),
        ],
    )(*args)


def _layer_tables(p, fin):
    W1e = p["phi_e"]["l1"]["W"]
    W1v = p["phi_v"]["l1"]["W"]
    Wd = jnp.concatenate([W1e[:fin], W1v[:fin]], axis=1)
    Ws = jnp.concatenate([W1e[fin:2 * fin], W1v[fin:2 * fin]], axis=1)
    return Wd, Ws


def kernel(x, params, edge_index, batch):
    src = edge_index[0]
    dst = edge_index[1]
    N, F = x.shape
    p1 = params["gnn1"]
    p2 = params["gnn2"]

    nsp = 4
    Eh = src.shape[0] // nsp
    halves = tuple((src[i * Eh:(i + 1) * Eh], dst[i * Eh:(i + 1) * Eh])
                   for i in range(nsp))

    def layer(P, Q, pp):
        parts = []
        for s_h, d_h in halves:
            pre_h, rel_h = _sc_gather(P, Q, s_h, d_h)
            msg_h = _edge(pre_h, rel_h, pp)
            ph = _sc_scatter([msg_h], [d_h], N)
            parts += [ph[:N], ph[_NP:_NP + N]]
        return parts

    Wd1, Ws1 = _layer_tables(p1, F)
    P1, Q1 = _prep(x, x, Wd1, Ws1)
    Wd2, Ws2 = _layer_tables(p2, 64)
    h1, P2, Q2 = _node(layer(P1, Q1, p1), x, p1, params["ln1"],
                       prep_w=(Wd2, Ws2))
    (h2,) = _node(layer(P2, Q2, p2), h1, p2, params["ln2"])

    pvp = jnp.pad(x[:, :4], ((0, 0), (0, 12)))
    s_out, _den, _pooled, mu2d, loss, lat2d = _pool(
        h2, pvp, batch.reshape(N, 1).astype(jnp.int32), params)
    latent = lat2d.reshape(8, 16, 32)
    mu = mu2d[:, :2].reshape(8, 16, 2)
    assign_losses = -jnp.sum(loss) / N
    return latent, s_out, assign_losses, mu
